# baseline (speedup 1.0000x reference)
import functools

import jax
import jax.numpy as jnp
from jax import lax
from jax.experimental import pallas as pl
from jax.experimental.pallas import tpu as pltpu
from jax.experimental.pallas import tpu_sc as plsc


def _relu(x):
    return jnp.maximum(x, 0.0)


def _gc_layer(support, u_feat, v_feat, W, b):
    C, U, I = support.shape
    F = u_feat.shape[1]
    H = W.shape[2]
    UB = 600
    NUB = U // UB
    b2 = b.reshape(1, H)

    def body(s_ref, uf_ref, vf_ref, w_ref, b_ref, outu_ref, outv_ref, tmpv_ref):
        i = pl.program_id(0)
        r = pl.program_id(1)
        w = w_ref[0]
        s = s_ref[0]

        @pl.when(i == 0)
        def _():
            tmpv_ref[pl.ds(r, 1)] = jnp.dot(
                vf_ref[...], w,
                preferred_element_type=jnp.float32).astype(jnp.bfloat16)[None]

        tv = tmpv_ref[pl.ds(r, 1)][0]
        tu = jnp.dot(uf_ref[...], w,
                     preferred_element_type=jnp.float32).astype(jnp.bfloat16)
        sb = s.astype(jnp.bfloat16)
        cu = jnp.dot(sb, tv, preferred_element_type=jnp.float32)
        cv = lax.dot_general(sb, tu, (((0,), (0,)), ((), ())),
                             preferred_element_type=jnp.float32)

        @pl.when(r == 0)
        def _():
            outu_ref[pl.ds(i * UB, UB), :] = cu

        @pl.when(r > 0)
        def _():
            outu_ref[pl.ds(i * UB, UB), :] += cu

        @pl.when((i == 0) & (r == 0))
        def _():
            outv_ref[...] = cv

        @pl.when((i > 0) | (r > 0))
        def _():
            outv_ref[...] += cv

        @pl.when((i == NUB - 1) & (r == C - 1))
        def _():
            bb = b_ref[...]
            outu_ref[...] = _relu(outu_ref[...] + bb)
            outv_ref[...] = _relu(outv_ref[...] + bb)

    return pl.pallas_call(
        body,
        grid=(NUB, C),
        in_specs=[
            pl.BlockSpec((1, UB, I), lambda i, r: (r, i, 0)),
            pl.BlockSpec((UB, F), lambda i, r: (i, 0)),
            pl.BlockSpec((I, F), lambda i, r: (0, 0)),
            pl.BlockSpec((1, F, H), lambda i, r: (r, 0, 0)),
            pl.BlockSpec((1, H), lambda i, r: (0, 0)),
        ],
        out_specs=[
            pl.BlockSpec((U, H), lambda i, r: (0, 0)),
            pl.BlockSpec((I, H), lambda i, r: (0, 0)),
        ],
        out_shape=[
            jax.ShapeDtypeStruct((U, H), jnp.float32),
            jax.ShapeDtypeStruct((I, H), jnp.float32),
        ],
        scratch_shapes=[pltpu.VMEM((C, I, H), jnp.bfloat16)],
        compiler_params=pltpu.CompilerParams(
            dimension_semantics=("arbitrary", "arbitrary")),
    )(support, u_feat, v_feat, W, b2)


def _dense_stage(u_z, v_z, u_side, v_side, Wu1, bu1, Wv1, bv1, Wu2, Wv2, P):
    U, Hz = u_z.shape
    I = v_z.shape[0]
    E = Wu1.shape[1]
    Ho = Wu2.shape[1]

    def body(uz_ref, vz_ref, us_ref, vs_ref, wu1_ref, bu1_ref, wv1_ref,
             bv1_ref, wu2_ref, wv2_ref, p0_ref, p1_ref, ua_ref, vh_ref):
        f32 = jnp.float32
        uf = _relu(jnp.dot(us_ref[...], wu1_ref[...],
                           preferred_element_type=f32) + bu1_ref[...])
        vf = _relu(jnp.dot(vs_ref[...], wv1_ref[...],
                           preferred_element_type=f32) + bv1_ref[...])
        uh = (
            jnp.dot(uz_ref[...], wu2_ref[0:Hz, :], preferred_element_type=f32)
            + jnp.dot(uf, wu2_ref[Hz:Hz + E, :], preferred_element_type=f32))
        vh = (
            jnp.dot(vz_ref[...], wv2_ref[0:Hz, :], preferred_element_type=f32)
            + jnp.dot(vf, wv2_ref[Hz:Hz + E, :], preferred_element_type=f32))
        ua_ref[...] = jnp.concatenate(
            [jnp.dot(uh, p0_ref[...], preferred_element_type=f32),
             jnp.dot(uh, p1_ref[...], preferred_element_type=f32)], axis=1)
        vh_ref[...] = jnp.concatenate(
            [vh, jnp.zeros((I, Ho), f32)], axis=1)

    return pl.pallas_call(
        body,
        out_shape=[
            jax.ShapeDtypeStruct((U, 2 * Ho), jnp.float32),
            jax.ShapeDtypeStruct((I, 2 * Ho), jnp.float32),
        ],
    )(u_z, v_z, u_side, v_side, Wu1, bu1.reshape(1, E), Wv1,
      bv1.reshape(1, E), Wu2, Wv2, P[0], P[1])


def _pair_gather(uh, vh, u_idx, v_idx):
    B = u_idx.shape[0]
    D = uh.shape[1]
    NW = 32
    CH = 128
    K = B // (NW * CH)
    u2 = u_idx.reshape(NW * K, CH)
    v2 = v_idx.reshape(NW * K, CH)
    mesh = plsc.VectorSubcoreMesh(core_axis_name="c", subcore_axis_name="s")

    @functools.partial(
        pl.kernel, mesh=mesh,
        out_type=[jax.ShapeDtypeStruct((B, D), jnp.float32),
                  jax.ShapeDtypeStruct((B, D), jnp.float32)],
        scratch_types=[
            pltpu.VMEM((K, CH), jnp.int32),
            pltpu.VMEM((K, CH), jnp.int32),
            pltpu.VMEM((K * CH, D), jnp.float32),
            pltpu.VMEM((K * CH, D), jnp.float32),
            pltpu.SemaphoreType.DMA,
        ],
    )
    def k(uh_hbm, u_hbm, vh_hbm, v_hbm, out_u, out_v,
          uix, vix, urows, vrows, sem):
        wid = lax.axis_index("s") * 2 + lax.axis_index("c")
        row0 = wid * K
        pltpu.sync_copy(u_hbm.at[pl.ds(row0, K)], uix)
        pltpu.sync_copy(v_hbm.at[pl.ds(row0, K)], vix)
        copies = []
        for j in range(K):
            copies.append(pltpu.async_copy(
                uh_hbm.at[uix.at[j]], urows.at[pl.ds(j * CH, CH)], sem))
            copies.append(pltpu.async_copy(
                vh_hbm.at[vix.at[j]], vrows.at[pl.ds(j * CH, CH)], sem))
        for c in copies:
            c.wait()
        base = row0 * CH
        pltpu.sync_copy(urows, out_u.at[pl.ds(base, K * CH)])
        pltpu.sync_copy(vrows, out_v.at[pl.ds(base, K * CH)])

    return k(uh, u2, vh, v2)


def _decode(U_g, V_g, a_comb):
    B, D2 = U_g.shape
    D = D2 // 2
    NCLS = a_comb.shape[1]
    NBLK = 4
    BB = B // NBLK
    a0 = a_comb[0:1]
    a1 = a_comb[1:2]
    U3 = U_g.reshape(B, 2, D)
    V3 = V_g.reshape(B, 2, D)

    def body(u_ref, v_ref, a0_ref, a1_ref, out_ref, mh_ref):
        f32 = jnp.float32
        vv = v_ref[:, 0, :]
        b0 = jnp.sum(u_ref[:, 0, :] * vv, axis=1, keepdims=True)
        b1 = jnp.sum(u_ref[:, 1, :] * vv, axis=1, keepdims=True)
        logits = b0 * a0_ref[...] + b1 * a1_ref[...]
        m = jnp.max(logits, axis=1, keepdims=True)
        e = jnp.exp(logits - m)
        sm = e / jnp.sum(e, axis=1, keepdims=True)
        cls = (lax.broadcasted_iota(jnp.int32, (1, NCLS), 1) + 1).astype(f32)
        out_ref[...] = logits
        mh_ref[...] = jnp.sum(sm * cls, axis=1, keepdims=True)

    outputs, mh = pl.pallas_call(
        body,
        grid=(NBLK,),
        in_specs=[
            pl.BlockSpec((BB, 2, D), lambda i: (i, 0, 0)),
            pl.BlockSpec((BB, 2, D), lambda i: (i, 0, 0)),
            pl.BlockSpec((1, NCLS), lambda i: (0, 0)),
            pl.BlockSpec((1, NCLS), lambda i: (0, 0)),
        ],
        out_specs=[
            pl.BlockSpec((BB, NCLS), lambda i: (i, 0)),
            pl.BlockSpec((BB, 1), lambda i: (i, 0)),
        ],
        out_shape=[
            jax.ShapeDtypeStruct((B, NCLS), jnp.float32),
            jax.ShapeDtypeStruct((B, 1), jnp.float32),
        ],
    )(U3, V3, a0, a1)
    return outputs, mh[:, 0]


def kernel(u, v, r_matrix, u_features, v_features, u_features_side,
           v_features_side, W1, b1, W2, b2, Wu1, bu1, Wv1, bv1, Wu2, Wv2,
           P, a_comb):
    u_z, v_z = _gc_layer(r_matrix, u_features, v_features, W1, b1)
    u_z, v_z = _gc_layer(r_matrix, u_z, v_z, W2, b2)
    ua, vh = _dense_stage(u_z, v_z, u_features_side, v_features_side,
                          Wu1, bu1, Wv1, bv1, Wu2, Wv2, P)
    U_g, V_g = _pair_gather(ua, vh, u, v)
    return _decode(U_g, V_g, a_comb)


# speedup vs baseline: 1.0707x; 1.0707x over previous
import functools

import jax
import jax.numpy as jnp
from jax import lax
from jax.experimental import pallas as pl
from jax.experimental.pallas import tpu as pltpu
from jax.experimental.pallas import tpu_sc as plsc


def _relu(x):
    return jnp.maximum(x, 0.0)


def _gc_layer(support, u_feat, v_feat, W, b):
    C, U, I = support.shape
    F = u_feat.shape[1]
    H = W.shape[2]
    UB = 600
    NUB = U // UB
    b2 = b.reshape(1, H)

    def body(s_ref, uf_ref, vf_ref, w_ref, b_ref, outu_ref, outv_ref,
             tmpv_ref, accv_ref):
        i = pl.program_id(0)
        r = pl.program_id(1)
        w = w_ref[0]
        s = s_ref[0]

        @pl.when(i == 0)
        def _():
            tmpv_ref[pl.ds(r, 1)] = jnp.dot(
                vf_ref[...], w,
                preferred_element_type=jnp.float32).astype(jnp.bfloat16)[None]

        tv = tmpv_ref[pl.ds(r, 1)][0]
        tu = jnp.dot(uf_ref[...], w,
                     preferred_element_type=jnp.float32).astype(jnp.bfloat16)
        sb = s.astype(jnp.bfloat16)
        cu = jnp.dot(sb, tv, preferred_element_type=jnp.float32)
        cvt = lax.dot_general(tu, sb, (((0,), (0,)), ((), ())),
                              preferred_element_type=jnp.float32)

        @pl.when(r == 0)
        def _():
            outu_ref[pl.ds(i * UB, UB), :] = cu

        @pl.when(r > 0)
        def _():
            outu_ref[pl.ds(i * UB, UB), :] += cu

        @pl.when((i == 0) & (r == 0))
        def _():
            accv_ref[...] = cvt

        @pl.when((i > 0) | (r > 0))
        def _():
            accv_ref[...] += cvt

        @pl.when((i == NUB - 1) & (r == C - 1))
        def _():
            bb = b_ref[...]
            outu_ref[...] = _relu(outu_ref[...] + bb)
            outv_ref[...] = _relu(accv_ref[...].T + bb)

    return pl.pallas_call(
        body,
        grid=(NUB, C),
        in_specs=[
            pl.BlockSpec((1, UB, I), lambda i, r: (r, i, 0)),
            pl.BlockSpec((UB, F), lambda i, r: (i, 0)),
            pl.BlockSpec((I, F), lambda i, r: (0, 0)),
            pl.BlockSpec((1, F, H), lambda i, r: (r, 0, 0)),
            pl.BlockSpec((1, H), lambda i, r: (0, 0)),
        ],
        out_specs=[
            pl.BlockSpec((U, H), lambda i, r: (0, 0)),
            pl.BlockSpec((I, H), lambda i, r: (0, 0)),
        ],
        out_shape=[
            jax.ShapeDtypeStruct((U, H), jnp.float32),
            jax.ShapeDtypeStruct((I, H), jnp.float32),
        ],
        scratch_shapes=[pltpu.VMEM((C, I, H), jnp.bfloat16),
                        pltpu.VMEM((H, I), jnp.float32)],
        compiler_params=pltpu.CompilerParams(
            dimension_semantics=("arbitrary", "arbitrary")),
    )(support, u_feat, v_feat, W, b2)


def _dense_stage(u_z, v_z, u_side, v_side, Wu1, bu1, Wv1, bv1, Wu2, Wv2, P):
    U, Hz = u_z.shape
    I = v_z.shape[0]
    E = Wu1.shape[1]
    Ho = Wu2.shape[1]

    def body(uz_ref, vz_ref, us_ref, vs_ref, wu1_ref, bu1_ref, wv1_ref,
             bv1_ref, wu2_ref, wv2_ref, p0_ref, p1_ref, ua_ref, vh_ref):
        f32 = jnp.float32
        uf = _relu(jnp.dot(us_ref[...], wu1_ref[...],
                           preferred_element_type=f32) + bu1_ref[...])
        vf = _relu(jnp.dot(vs_ref[...], wv1_ref[...],
                           preferred_element_type=f32) + bv1_ref[...])
        uh = (
            jnp.dot(uz_ref[...], wu2_ref[0:Hz, :], preferred_element_type=f32)
            + jnp.dot(uf, wu2_ref[Hz:Hz + E, :], preferred_element_type=f32))
        vh = (
            jnp.dot(vz_ref[...], wv2_ref[0:Hz, :], preferred_element_type=f32)
            + jnp.dot(vf, wv2_ref[Hz:Hz + E, :], preferred_element_type=f32))
        ua_ref[...] = jnp.concatenate(
            [jnp.dot(uh, p0_ref[...], preferred_element_type=f32),
             jnp.dot(uh, p1_ref[...], preferred_element_type=f32)], axis=1)
        vh_ref[...] = jnp.concatenate(
            [vh, jnp.zeros((I, Ho), f32)], axis=1)

    return pl.pallas_call(
        body,
        out_shape=[
            jax.ShapeDtypeStruct((U, 2 * Ho), jnp.float32),
            jax.ShapeDtypeStruct((I, 2 * Ho), jnp.float32),
        ],
    )(u_z, v_z, u_side, v_side, Wu1, bu1.reshape(1, E), Wv1,
      bv1.reshape(1, E), Wu2, Wv2, P[0], P[1])


def _pair_gather(uh, vh, u_idx, v_idx):
    B = u_idx.shape[0]
    D = uh.shape[1]
    NW = 32
    CH = 128
    K = B // (NW * CH)
    u2 = u_idx.reshape(NW * K, CH)
    v2 = v_idx.reshape(NW * K, CH)
    mesh = plsc.VectorSubcoreMesh(core_axis_name="c", subcore_axis_name="s")

    @functools.partial(
        pl.kernel, mesh=mesh,
        out_type=[jax.ShapeDtypeStruct((B, D), jnp.float32),
                  jax.ShapeDtypeStruct((B, D), jnp.float32)],
        scratch_types=[
            pltpu.VMEM((K, CH), jnp.int32),
            pltpu.VMEM((K, CH), jnp.int32),
            pltpu.VMEM((K * CH, D), jnp.float32),
            pltpu.VMEM((K * CH, D), jnp.float32),
            pltpu.SemaphoreType.DMA,
        ],
    )
    def k(uh_hbm, u_hbm, vh_hbm, v_hbm, out_u, out_v,
          uix, vix, urows, vrows, sem):
        wid = lax.axis_index("s") * 2 + lax.axis_index("c")
        row0 = wid * K
        pltpu.sync_copy(u_hbm.at[pl.ds(row0, K)], uix)
        pltpu.sync_copy(v_hbm.at[pl.ds(row0, K)], vix)
        copies = []
        for j in range(K):
            copies.append(pltpu.async_copy(
                uh_hbm.at[uix.at[j]], urows.at[pl.ds(j * CH, CH)], sem))
            copies.append(pltpu.async_copy(
                vh_hbm.at[vix.at[j]], vrows.at[pl.ds(j * CH, CH)], sem))
        for c in copies:
            c.wait()
        base = row0 * CH
        pltpu.sync_copy(urows, out_u.at[pl.ds(base, K * CH)])
        pltpu.sync_copy(vrows, out_v.at[pl.ds(base, K * CH)])

    return k(uh, u2, vh, v2)


def _decode(U_g, V_g, a_comb):
    B, D2 = U_g.shape
    D = D2 // 2
    NCLS = a_comb.shape[1]
    NBLK = 4
    BB = B // NBLK
    a0 = a_comb[0:1]
    a1 = a_comb[1:2]
    U3 = U_g.reshape(B, 2, D)
    V3 = V_g.reshape(B, 2, D)

    def body(u_ref, v_ref, a0_ref, a1_ref, out_ref, mh_ref):
        f32 = jnp.float32
        vv = v_ref[:, 0, :]
        b0 = jnp.sum(u_ref[:, 0, :] * vv, axis=1, keepdims=True)
        b1 = jnp.sum(u_ref[:, 1, :] * vv, axis=1, keepdims=True)
        logits = b0 * a0_ref[...] + b1 * a1_ref[...]
        m = jnp.max(logits, axis=1, keepdims=True)
        e = jnp.exp(logits - m)
        sm = e / jnp.sum(e, axis=1, keepdims=True)
        cls = (lax.broadcasted_iota(jnp.int32, (1, NCLS), 1) + 1).astype(f32)
        out_ref[...] = logits
        mh_ref[...] = jnp.sum(sm * cls, axis=1, keepdims=True)

    outputs, mh = pl.pallas_call(
        body,
        grid=(NBLK,),
        in_specs=[
            pl.BlockSpec((BB, 2, D), lambda i: (i, 0, 0)),
            pl.BlockSpec((BB, 2, D), lambda i: (i, 0, 0)),
            pl.BlockSpec((1, NCLS), lambda i: (0, 0)),
            pl.BlockSpec((1, NCLS), lambda i: (0, 0)),
        ],
        out_specs=[
            pl.BlockSpec((BB, NCLS), lambda i: (i, 0)),
            pl.BlockSpec((BB, 1), lambda i: (i, 0)),
        ],
        out_shape=[
            jax.ShapeDtypeStruct((B, NCLS), jnp.float32),
            jax.ShapeDtypeStruct((B, 1), jnp.float32),
        ],
    )(U3, V3, a0, a1)
    return outputs, mh[:, 0]


def kernel(u, v, r_matrix, u_features, v_features, u_features_side,
           v_features_side, W1, b1, W2, b2, Wu1, bu1, Wv1, bv1, Wu2, Wv2,
           P, a_comb):
    u_z, v_z = _gc_layer(r_matrix, u_features, v_features, W1, b1)
    u_z, v_z = _gc_layer(r_matrix, u_z, v_z, W2, b2)
    ua, vh = _dense_stage(u_z, v_z, u_features_side, v_features_side,
                          Wu1, bu1, Wv1, bv1, Wu2, Wv2, P)
    U_g, V_g = _pair_gather(ua, vh, u, v)
    return _decode(U_g, V_g, a_comb)
